# baseline (speedup 1.0000x reference)
import functools

import jax
import jax.numpy as jnp
from jax import lax
from jax.experimental import pallas as pl
from jax.experimental.pallas import tpu as pltpu
from jax.experimental.pallas import tpu_sc as plsc

N_NODES = 10000
N_EDGES = 160000
IN_DIM = 256
OUT_DIM = 256

NC = 2
NS = 16
L = 16

HALF = OUT_DIM // 2
CHUNK = 64
T = 160
E_PAD = NS * T * CHUNK
N_BUF = 5
N_IBUF = 10
LEAD = 3
N_PAD = 10240
ROWS_PER_TILE = N_PAD // NS
ROW_CHUNK = 64
N_ROW_CHUNKS = ROWS_PER_TILE // ROW_CHUNK

_GATHER_DN = lax.GatherDimensionNumbers(
    offset_dims=(), collapsed_slice_dims=(0,), start_index_map=(0,))


def _matmul_body(x_ref, w_ref, out_ref):
    out_ref[0] = jnp.dot(x_ref[...], w_ref[...],
                         preferred_element_type=jnp.float32)


def _tc_matmul(x, W):
    n, k = x.shape
    bn = 1000
    return pl.pallas_call(
        _matmul_body,
        grid=(NC, n // bn),
        in_specs=[
            pl.BlockSpec((bn, k), lambda h, r: (r, 0)),
            pl.BlockSpec((k, HALF), lambda h, r: (0, h)),
        ],
        out_specs=pl.BlockSpec((1, bn, HALF), lambda h, r: (h, r, 0)),
        out_shape=jax.ShapeDtypeStruct((NC, n, HALF), jnp.float32),
    )(x, W)


_sc_mesh = plsc.VectorSubcoreMesh(core_axis_name="c", subcore_axis_name="s")


@functools.partial(
    pl.kernel,
    out_type=jax.ShapeDtypeStruct((NC, N_PAD, HALF), jnp.float32),
    mesh=_sc_mesh,
    scratch_types=[
        pltpu.VMEM((N_IBUF, CHUNK), jnp.int32),
        pltpu.VMEM((N_IBUF, CHUNK), jnp.int32),
        pltpu.VMEM((N_IBUF, CHUNK), jnp.float32),
        pltpu.VMEM((N_BUF, CHUNK, HALF), jnp.float32),
        pltpu.VMEM_SHARED((N_PAD, HALF), jnp.float32),
    ] + [pltpu.SemaphoreType.DMA] * (N_BUF + N_BUF + N_IBUF),
)
def _sc_agg(hs_hbm, col_hbm, row_hbm, adj_hbm, out_hbm,
            colring, rowring, adjring, msgbuf, acc, *sems):
    core = lax.axis_index("c")
    sub = lax.axis_index("s")
    gsem = sems[:N_BUF]
    ssem = sems[N_BUF:2 * N_BUF]
    isem = sems[2 * N_BUF:]

    def _idx_start(t, b8):
        pltpu.async_copy(col_hbm.at[core, sub, t], colring.at[b8], isem[b8])
        pltpu.async_copy(row_hbm.at[sub, t], rowring.at[b8], isem[b8])
        pltpu.async_copy(adj_hbm.at[sub, t], adjring.at[b8], isem[b8])

    def _idx_wait(t, b8):
        pltpu.make_async_copy(col_hbm.at[core, sub, t], colring.at[b8],
                              isem[b8]).wait()
        pltpu.make_async_copy(row_hbm.at[sub, t], rowring.at[b8],
                              isem[b8]).wait()
        pltpu.make_async_copy(adj_hbm.at[sub, t], adjring.at[b8],
                              isem[b8]).wait()

    def _gather_start(b8, b4):
        pltpu.async_copy(hs_hbm.at[colring.at[b8]], msgbuf.at[b4], gsem[b4])

    def _gather_wait(b8, b4):
        pltpu.make_async_copy(hs_hbm.at[colring.at[b8]], msgbuf.at[b4],
                              gsem[b4]).wait()

    def _scatter_start(b8, b4):
        pltpu.async_copy(msgbuf.at[b4], acc.at[rowring.at[b8]], ssem[b4],
                         add=True)

    def _scatter_wait(b8, b4):
        pltpu.make_async_copy(msgbuf.at[b4], acc.at[rowring.at[b8]],
                              ssem[b4]).wait()

    zeros = jnp.zeros((L,), jnp.float32)

    def _zero_row(r, _):
        for f in range(HALF // L):
            msgbuf[0, r, pl.ds(f * L, L)] = zeros
        return 0

    lax.fori_loop(0, ROW_CHUNK, _zero_row, 0)
    row_base = sub * ROWS_PER_TILE
    for kk in range(N_ROW_CHUNKS):
        pltpu.sync_copy(msgbuf.at[0], acc.at[pl.ds(row_base + kk * ROW_CHUNK,
                                                   ROW_CHUNK)])

    for t in range(N_BUF):
        _idx_start(t, t)
    for t in range(LEAD):
        _idx_wait(t, t)
        _gather_start(t, t)
    plsc.subcore_barrier()

    def _scale(b8, b4):
        def _scale16(i, _):
            av = adjring[b8, pl.ds(pl.multiple_of(i * L, 8), L)]
            for j in range(L):
                a = lax.gather(av, jnp.full((L, 1), j, jnp.int32),
                               _GATHER_DN, slice_sizes=(1,),
                               mode=lax.GatherScatterMode.PROMISE_IN_BOUNDS)
                for f in range(HALF // L):
                    sl = pl.ds(f * L, L)
                    msgbuf[b4, i * L + j, sl] = msgbuf[b4, i * L + j, sl] * a
            return 0

        lax.fori_loop(0, CHUNK // L, _scale16, 0)

    def _visit(t, v):
        b4 = v % N_BUF
        b8 = v % N_IBUF
        _gather_wait(b8, b4)
        _scale(b8, b4)
        _scatter_start(b8, b4)

        nb4 = (v + LEAD) % N_BUF
        nb8 = (v + LEAD) % N_IBUF

        @pl.when(t + LEAD < T)
        def _():
            @pl.when(t >= N_BUF - LEAD)
            def _():
                _scatter_wait(nb8, nb4)
            _idx_wait(t + LEAD, nb8)
            _gather_start(nb8, nb4)

        @pl.when(t + N_BUF < T)
        def _():
            _idx_start(t + N_BUF, (v + N_BUF) % N_IBUF)

    def _round(k, _):
        for v in range(N_IBUF):
            _visit(k * N_IBUF + v, v)
        return 0

    lax.fori_loop(0, T // N_IBUF, _round, 0)

    for t in range(T - N_BUF, T):
        _scatter_wait(t % N_IBUF, t % N_BUF)
    plsc.subcore_barrier()

    for kk in range(N_ROW_CHUNKS):
        r0 = row_base + kk * ROW_CHUNK
        pltpu.sync_copy(acc.at[pl.ds(r0, ROW_CHUNK)], msgbuf.at[0])

        def _relu(r, _):
            for f in range(HALF // L):
                sl = pl.ds(f * L, L)
                msgbuf[0, r, sl] = jnp.maximum(msgbuf[0, r, sl], 0.0)
            return 0

        lax.fori_loop(0, ROW_CHUNK, _relu, 0)
        pltpu.sync_copy(msgbuf.at[0], out_hbm.at[core, pl.ds(r0, ROW_CHUNK)])


def kernel(x, edge_index, adj_values, features_nonzero, W):
    del features_nonzero
    ei = edge_index.astype(jnp.int32)
    pad = E_PAD - N_EDGES
    row = jnp.concatenate([ei[0], jnp.zeros((pad,), jnp.int32)])
    col = jnp.concatenate([ei[1], jnp.zeros((pad,), jnp.int32)])
    adj = jnp.concatenate([adj_values.astype(jnp.float32),
                           jnp.zeros((pad,), jnp.float32)])
    col2 = (col[None, :] +
            jnp.array([[0], [N_NODES]], jnp.int32)).reshape(NC, NS, T, CHUNK)
    row3 = row.reshape(NS, T, CHUNK)
    adj3 = adj.reshape(NS, T, CHUNK)
    hs = _tc_matmul(x.astype(jnp.float32), W.astype(jnp.float32))
    hs_flat = hs.reshape(NC * N_NODES, HALF)
    out2 = _sc_agg(hs_flat, col2, row3, adj3)
    return out2[:, :N_NODES].transpose(1, 0, 2).reshape(N_NODES, OUT_DIM)


# speedup vs baseline: 1.0811x; 1.0811x over previous
import functools

import jax
import jax.numpy as jnp
from jax import lax
from jax.experimental import pallas as pl
from jax.experimental.pallas import tpu as pltpu
from jax.experimental.pallas import tpu_sc as plsc

N_NODES = 10000
N_EDGES = 160000
IN_DIM = 256
OUT_DIM = 256

NC = 2
NS = 16
L = 16

HALF = OUT_DIM // 2
CHUNK = 64
T = 160
E_PAD = NS * T * CHUNK
N_BUF = 5
N_IBUF = 10
LEAD = 3
N_PAD = 10240
ROWS_PER_TILE = N_PAD // NS
ROW_CHUNK = 64
N_ROW_CHUNKS = ROWS_PER_TILE // ROW_CHUNK

_GATHER_DN = lax.GatherDimensionNumbers(
    offset_dims=(), collapsed_slice_dims=(0,), start_index_map=(0,))


def _matmul_body(x_ref, w_ref, out_ref):
    out_ref[0] = jnp.dot(x_ref[...], w_ref[...],
                         preferred_element_type=jnp.float32)


def _tc_matmul(x, W):
    n, k = x.shape
    bn = 1000
    return pl.pallas_call(
        _matmul_body,
        grid=(NC, n // bn),
        in_specs=[
            pl.BlockSpec((bn, k), lambda h, r: (r, 0)),
            pl.BlockSpec((k, HALF), lambda h, r: (0, h)),
        ],
        out_specs=pl.BlockSpec((1, bn, HALF), lambda h, r: (h, r, 0)),
        out_shape=jax.ShapeDtypeStruct((NC, n, HALF), jnp.float32),
    )(x, W)


_sc_mesh = plsc.VectorSubcoreMesh(core_axis_name="c", subcore_axis_name="s")


@functools.partial(
    pl.kernel,
    out_type=jax.ShapeDtypeStruct((N_PAD, OUT_DIM), jnp.float32),
    mesh=_sc_mesh,
    scratch_types=[
        pltpu.VMEM((N_IBUF, CHUNK), jnp.int32),
        pltpu.VMEM((N_IBUF, CHUNK), jnp.int32),
        pltpu.VMEM((N_IBUF, CHUNK), jnp.float32),
        pltpu.VMEM((N_BUF, CHUNK, HALF), jnp.float32),
        pltpu.VMEM_SHARED((N_PAD, HALF), jnp.float32),
    ] + [pltpu.SemaphoreType.DMA] * (N_BUF + N_BUF + N_IBUF),
)
def _sc_agg(hs_hbm, col_hbm, row_hbm, adj_hbm, out_hbm,
            colring, rowring, adjring, msgbuf, acc, *sems):
    core = lax.axis_index("c")
    sub = lax.axis_index("s")
    gsem = sems[:N_BUF]
    ssem = sems[N_BUF:2 * N_BUF]
    isem = sems[2 * N_BUF:]

    def _idx_start(t, b8):
        pltpu.async_copy(col_hbm.at[core, sub, t], colring.at[b8], isem[b8])
        pltpu.async_copy(row_hbm.at[sub, t], rowring.at[b8], isem[b8])
        pltpu.async_copy(adj_hbm.at[sub, t], adjring.at[b8], isem[b8])

    def _idx_wait(t, b8):
        pltpu.make_async_copy(col_hbm.at[core, sub, t], colring.at[b8],
                              isem[b8]).wait()
        pltpu.make_async_copy(row_hbm.at[sub, t], rowring.at[b8],
                              isem[b8]).wait()
        pltpu.make_async_copy(adj_hbm.at[sub, t], adjring.at[b8],
                              isem[b8]).wait()

    def _gather_start(b8, b4):
        pltpu.async_copy(hs_hbm.at[colring.at[b8]], msgbuf.at[b4], gsem[b4])

    def _gather_wait(b8, b4):
        pltpu.make_async_copy(hs_hbm.at[colring.at[b8]], msgbuf.at[b4],
                              gsem[b4]).wait()

    def _scatter_start(b8, b4):
        pltpu.async_copy(msgbuf.at[b4], acc.at[rowring.at[b8]], ssem[b4],
                         add=True)

    def _scatter_wait(b8, b4):
        pltpu.make_async_copy(msgbuf.at[b4], acc.at[rowring.at[b8]],
                              ssem[b4]).wait()

    zeros = jnp.zeros((L,), jnp.float32)

    def _zero_row(r, _):
        for f in range(HALF // L):
            msgbuf[0, r, pl.ds(f * L, L)] = zeros
        return 0

    lax.fori_loop(0, ROW_CHUNK, _zero_row, 0)
    row_base = sub * ROWS_PER_TILE
    for kk in range(N_ROW_CHUNKS):
        pltpu.sync_copy(msgbuf.at[0], acc.at[pl.ds(row_base + kk * ROW_CHUNK,
                                                   ROW_CHUNK)])

    for t in range(N_BUF):
        _idx_start(t, t)
    for t in range(LEAD):
        _idx_wait(t, t)
        _gather_start(t, t)
    plsc.subcore_barrier()

    def _scale(b8, b4):
        def _scale16(i, _):
            av = adjring[b8, pl.ds(pl.multiple_of(i * L, 8), L)]
            for j in range(L):
                a = lax.gather(av, jnp.full((L, 1), j, jnp.int32),
                               _GATHER_DN, slice_sizes=(1,),
                               mode=lax.GatherScatterMode.PROMISE_IN_BOUNDS)
                for f in range(HALF // L):
                    sl = pl.ds(f * L, L)
                    msgbuf[b4, i * L + j, sl] = msgbuf[b4, i * L + j, sl] * a
            return 0

        lax.fori_loop(0, CHUNK // L, _scale16, 0)

    def _visit(t, v):
        b4 = v % N_BUF
        b8 = v % N_IBUF
        _gather_wait(b8, b4)
        _scale(b8, b4)
        _scatter_start(b8, b4)

        nb4 = (v + LEAD) % N_BUF
        nb8 = (v + LEAD) % N_IBUF

        @pl.when(t + LEAD < T)
        def _():
            @pl.when(t >= N_BUF - LEAD)
            def _():
                _scatter_wait(nb8, nb4)
            _idx_wait(t + LEAD, nb8)
            _gather_start(nb8, nb4)

        @pl.when(t + N_BUF < T)
        def _():
            _idx_start(t + N_BUF, (v + N_BUF) % N_IBUF)

    def _round(k, _):
        for v in range(N_IBUF):
            _visit(k * N_IBUF + v, v)
        return 0

    lax.fori_loop(0, T // N_IBUF, _round, 0)

    for t in range(T - N_BUF, T):
        _scatter_wait(t % N_IBUF, t % N_BUF)
    plsc.subcore_barrier()

    for kk in range(N_ROW_CHUNKS):
        r0 = row_base + kk * ROW_CHUNK
        pltpu.sync_copy(acc.at[pl.ds(r0, ROW_CHUNK)], msgbuf.at[0])

        def _relu(r, _):
            for f in range(HALF // L):
                sl = pl.ds(f * L, L)
                msgbuf[0, r, sl] = jnp.maximum(msgbuf[0, r, sl], 0.0)
            return 0

        lax.fori_loop(0, ROW_CHUNK, _relu, 0)
        c0 = pl.multiple_of(core * HALF, 128)
        pltpu.sync_copy(msgbuf.at[0],
                        out_hbm.at[pl.ds(r0, ROW_CHUNK), pl.ds(c0, HALF)])


def kernel(x, edge_index, adj_values, features_nonzero, W):
    del features_nonzero
    ei = edge_index.astype(jnp.int32)
    pad = E_PAD - N_EDGES
    row = jnp.concatenate([ei[0], jnp.zeros((pad,), jnp.int32)])
    col = jnp.concatenate([ei[1], jnp.zeros((pad,), jnp.int32)])
    adj = jnp.concatenate([adj_values.astype(jnp.float32),
                           jnp.zeros((pad,), jnp.float32)])
    col2 = (col[None, :] +
            jnp.array([[0], [N_NODES]], jnp.int32)).reshape(NC, NS, T, CHUNK)
    row3 = row.reshape(NS, T, CHUNK)
    adj3 = adj.reshape(NS, T, CHUNK)
    hs = _tc_matmul(x.astype(jnp.float32), W.astype(jnp.float32))
    hs_flat = hs.reshape(NC * N_NODES, HALF)
    out2 = _sc_agg(hs_flat, col2, row3, adj3)
    return out2[:N_NODES]
